# Initial kernel scaffold; baseline (speedup 1.0000x reference)
import jax
import jax.numpy as jnp
from jax.experimental import pallas as pl


def kernel(x, edge_index, W1, a1s, a1d, b1, W2, a2s, a2d, b2):
    raise NotImplementedError("write your pallas kernel here")


# speedup vs baseline: 72.8722x; 72.8722x over previous
import functools

import numpy as np
import jax
import jax.numpy as jnp
from jax import lax
from jax.experimental import pallas as pl
from jax.experimental.pallas import tpu as pltpu
from jax.experimental.pallas import tpu_sc as plsc

_NEG = 0.2
_NC, _NS, _L = 2, 16, 16
_NW = _NC * _NS
_CH = 80


def _lrelu(t):
    return jnp.maximum(t, _NEG * t)


def _prep_body(x_ref, wcat_ref, hsrc_ref, adp_ref, m1_ref):
    c1 = hsrc_ref.shape[1] - _L
    xw = jnp.dot(x_ref[...], wcat_ref[...], preferred_element_type=jnp.float32)
    asp = xw[:, c1:c1 + _L]
    adp = xw[:, c1 + _L:]
    hsrc_ref[:, :c1] = xw[:, :c1]
    hsrc_ref[:, c1:] = asp
    adp_ref[...] = adp
    m1_ref[...] = (jnp.max(asp, axis=0, keepdims=True)
                   + jnp.max(adp, axis=0, keepdims=True))


def _mid_body(p0_ref, p1_ref, hsrc_ref, adp_ref, m1_ref, b1p_ref, wcat2_ref,
              l2s_ref, l2d_ref, m2_ref):
    c1 = hsrc_ref.shape[1] - _L
    tot = p0_ref[...] + p1_ref[...]
    asp = hsrc_ref[:, c1:]
    wself = jnp.exp(_lrelu(asp + adp_ref[...]) - m1_ref[...])
    rr = lax.broadcasted_iota(jnp.int32, (_L, c1), 0)
    cc = lax.broadcasted_iota(jnp.int32, (_L, c1), 1)
    tile16 = (cc % _L == rr).astype(jnp.float32)
    wrep = jnp.dot(wself, tile16, preferred_element_type=jnp.float32)
    num = tot[:, :c1] + wrep * hsrc_ref[:, :c1]
    den = tot[:, c1:] + wself
    denrep = jnp.dot(den, tile16, preferred_element_type=jnp.float32)
    xh = num / (denrep + 1e-16) + b1p_ref[...]
    z = jnp.dot(xh, wcat2_ref[...], preferred_element_type=jnp.float32)
    h2 = z[:, :_L]
    as2 = z[:, _L:2 * _L]
    ad2 = z[:, 2 * _L:]
    m2_ref[...] = (jnp.max(as2, axis=0, keepdims=True)
                   + jnp.max(ad2, axis=0, keepdims=True))
    lanes = lax.broadcasted_iota(jnp.int32, (1, _L), 1)
    e0 = (lanes == 0).astype(jnp.float32)
    e1 = (lanes == 1).astype(jnp.float32)
    l2s_ref[:, :_L] = h2 * e0 + e1
    l2s_ref[:, _L:] = as2
    l2d_ref[...] = ad2


def _fin_body(q0_ref, q1_ref, l2s_ref, l2d_ref, m2_ref, b2_ref, out_ref):
    s1 = l2s_ref[:, :_L]
    s0 = l2s_ref[:, _L:]
    w = jnp.exp(_lrelu(s0 + l2d_ref[...]) - m2_ref[...])
    tot = (q0_ref[:, :_L] + q1_ref[:, :_L] + w * s1)
    rr = lax.broadcasted_iota(jnp.int32, (_L, _L), 0)
    sel0 = (rr == 0).astype(jnp.float32)
    sel1 = (rr == 1).astype(jnp.float32)
    numb = jnp.dot(tot, sel0, preferred_element_type=jnp.float32)
    denb = jnp.dot(tot, sel1, preferred_element_type=jnp.float32)
    o = numb / (denb + 1e-16) + b2_ref[...]
    out_ref[...] = jnp.where(o > 0, o, jnp.exp(jnp.minimum(o, 0.0)) - 1.0)


def _tc(body, out_shape, *ins):
    return pl.pallas_call(body, out_shape=out_shape)(*ins)


def _make_edge_pass(n_nodes, n_edges, row_w, n_vregs_payload):
    ew = n_edges // _NW
    nch = ew // _CH
    assert ew % _CH == 0
    zr = 128
    npad = -(-n_nodes // (zr * _NS)) * (zr * _NS)
    rpt = npad // _NS
    mesh = plsc.VectorSubcoreMesh(core_axis_name="c", subcore_axis_name="s",
                                  num_cores=_NC, num_subcores=_NS)

    @functools.partial(
        pl.kernel,
        out_type=jax.ShapeDtypeStruct((_NC, npad, row_w), jnp.float32),
        mesh=mesh,
        compiler_params=pltpu.CompilerParams(use_tc_tiling_on_sc=False),
        scratch_types=[
            pltpu.VMEM((_CH,), jnp.int32),
            pltpu.VMEM((_CH,), jnp.int32),
            pltpu.VMEM((_CH, row_w), jnp.float32),
            pltpu.VMEM((_CH, _L), jnp.float32),
            pltpu.VMEM((_L,), jnp.float32),
            pltpu.VMEM((zr, row_w), jnp.float32),
            pltpu.VMEM_SHARED((npad, row_w), jnp.float32),
            pltpu.SemaphoreType.DMA,
        ],
    )
    def k(src_hbm, dst_hbm, pack_hbm, adp_hbm, m_hbm, out_hbm,
          idxs_v, idxd_v, rows_v, ad_v, m_v, zero_v, acc_sh, sem):
        cid = lax.axis_index("c")
        sid = lax.axis_index("s")

        def zrow(r, carry):
            for kk in range(row_w // _L):
                zero_v[r, pl.ds(kk * _L, _L)] = jnp.zeros((_L,), jnp.float32)
            return carry
        lax.fori_loop(0, zr, zrow, 0)

        def zcopy(b, carry):
            pltpu.sync_copy(zero_v,
                            acc_sh.at[pl.ds(sid * rpt + b * zr, zr)])
            return carry
        lax.fori_loop(0, rpt // zr, zcopy, 0)
        pltpu.sync_copy(m_hbm, m_v)
        plsc.subcore_barrier()

        mvec = m_v[...]
        ebase = (cid * _NS + sid) * ew

        def chunk(ci, carry):
            base = ebase + ci * _CH
            pltpu.sync_copy(src_hbm.at[pl.ds(base, _CH)], idxs_v)
            pltpu.sync_copy(dst_hbm.at[pl.ds(base, _CH)], idxd_v)
            d1 = pltpu.async_copy(pack_hbm.at[idxs_v], rows_v, sem)
            d1.wait()
            d2 = pltpu.async_copy(adp_hbm.at[idxd_v], ad_v, sem)
            d2.wait()

            def edge(e, c2):
                asv = rows_v[e, pl.ds(row_w - _L, _L)]
                t = asv + ad_v[e, pl.ds(0, _L)]
                w = jnp.exp(jnp.maximum(t, _NEG * t) - mvec)
                for kk in range(n_vregs_payload):
                    hv = rows_v[e, pl.ds(kk * _L, _L)]
                    rows_v[e, pl.ds(kk * _L, _L)] = hv * w
                rows_v[e, pl.ds(row_w - _L, _L)] = w
                return c2
            lax.fori_loop(0, _CH, edge, 0)
            pltpu.sync_copy(rows_v, acc_sh.at[idxd_v], add=True)
            return carry
        lax.fori_loop(0, nch, chunk, 0)

        plsc.subcore_barrier()
        pltpu.sync_copy(acc_sh.at[pl.ds(sid * rpt, rpt)],
                        out_hbm.at[cid, pl.ds(sid * rpt, rpt)])
    return k


def kernel(x, edge_index, W1, a1s, a1d, b1, W2, a2s, a2d, b2):
    n, f_in = x.shape
    e = edge_index.shape[1]
    h, f_hid = a1s.shape
    c1 = h * f_hid
    p1 = c1 + _L
    f32 = jnp.float32

    hh, ff = np.meshgrid(np.arange(h), np.arange(f_hid), indexing="ij")
    dest = ((ff // 2) * _L + (ff % 2) * h + hh).reshape(-1)
    inv = np.empty((c1,), np.int64)
    inv[dest] = np.arange(c1)

    w1flat = jnp.transpose(W1, (1, 0, 2)).reshape(f_in, c1)
    w1perm = w1flat[:, inv]
    avs = jnp.einsum("hif,hf->ih", W1, a1s)
    avd = jnp.einsum("hif,hf->ih", W1, a1d)
    wcat = jnp.concatenate(
        [w1perm, jnp.tile(avs, (1, 2)), jnp.tile(avd, (1, 2))], axis=1)

    b1p = b1.reshape(1, c1)[:, inv]
    w2p = W2[inv, :]
    w2rep = jnp.tile(w2p, (1, _L))
    wcat2 = jnp.concatenate(
        [w2rep, w2rep * a2s[0], w2rep * a2d[0]], axis=1)

    src = edge_index[0]
    dst = edge_index[1]

    hsrc, adp, m1 = _tc(
        _prep_body,
        (jax.ShapeDtypeStruct((n, p1), f32),
         jax.ShapeDtypeStruct((n, _L), f32),
         jax.ShapeDtypeStruct((1, _L), f32)),
        x, wcat)

    part1 = _make_edge_pass(n, e, p1, c1 // _L)(
        src, dst, hsrc, adp, m1.reshape(_L))

    l2s, l2d, m2 = _tc(
        _mid_body,
        (jax.ShapeDtypeStruct((n, 2 * _L), f32),
         jax.ShapeDtypeStruct((n, _L), f32),
         jax.ShapeDtypeStruct((1, _L), f32)),
        part1[0, :n], part1[1, :n], hsrc, adp, m1, b1p, wcat2)

    part2 = _make_edge_pass(n, e, 2 * _L, 1)(
        src, dst, l2s, l2d, m2.reshape(_L))

    out16 = _tc(
        _fin_body,
        jax.ShapeDtypeStruct((n, _L), f32),
        part2[0, :n], part2[1, :n], l2s, l2d, m2, b2.reshape(1, 1))

    return out16[:, :1]


# speedup vs baseline: 121.3063x; 1.6646x over previous
import functools

import numpy as np
import jax
import jax.numpy as jnp
from jax import lax
from jax.experimental import pallas as pl
from jax.experimental.pallas import tpu as pltpu
from jax.experimental.pallas import tpu_sc as plsc

_NEG = 0.2
_NC, _NS, _L = 2, 16, 16
_NW = _NC * _NS


def _lrelu(t):
    return jnp.maximum(t, _NEG * t)


def _prep_body(x_ref, wcat_ref, hsrc_ref, adp_ref, m1_ref):
    c1 = hsrc_ref.shape[1] - _L
    xw = jnp.dot(x_ref[...], wcat_ref[...], preferred_element_type=jnp.float32)
    asp = xw[:, c1:c1 + _L]
    adp = xw[:, c1 + _L:]
    hsrc_ref[:, :c1] = xw[:, :c1]
    hsrc_ref[:, c1:] = asp
    adp_ref[...] = adp
    m1_ref[...] = (jnp.max(asp, axis=0, keepdims=True)
                   + jnp.max(adp, axis=0, keepdims=True))


def _mid_body(p0_ref, p1_ref, hsrc_ref, adp_ref, m1_ref, b1p_ref, wcat2_ref,
              l2s_ref, l2d_ref, m2_ref):
    c1 = hsrc_ref.shape[1] - _L
    tot = p0_ref[...] + p1_ref[...]
    asp = hsrc_ref[:, c1:]
    wself = jnp.exp(_lrelu(asp + adp_ref[...]) - m1_ref[...])
    rr = lax.broadcasted_iota(jnp.int32, (_L, c1), 0)
    cc = lax.broadcasted_iota(jnp.int32, (_L, c1), 1)
    tile16 = (cc % _L == rr).astype(jnp.float32)
    wrep = jnp.dot(wself, tile16, preferred_element_type=jnp.float32)
    num = tot[:, :c1] + wrep * hsrc_ref[:, :c1]
    den = tot[:, c1:] + wself
    denrep = jnp.dot(den, tile16, preferred_element_type=jnp.float32)
    xh = num / (denrep + 1e-16) + b1p_ref[...]
    z = jnp.dot(xh, wcat2_ref[...], preferred_element_type=jnp.float32)
    h2 = z[:, :_L]
    as2 = z[:, _L:2 * _L]
    ad2 = z[:, 2 * _L:]
    m2_ref[...] = (jnp.max(as2, axis=0, keepdims=True)
                   + jnp.max(ad2, axis=0, keepdims=True))
    lanes = lax.broadcasted_iota(jnp.int32, (1, _L), 1)
    e0 = (lanes == 0).astype(jnp.float32)
    e1 = (lanes == 1).astype(jnp.float32)
    l2s_ref[:, :_L] = h2 * e0 + e1
    l2s_ref[:, _L:] = as2
    l2d_ref[...] = ad2


def _fin_body(q0_ref, q1_ref, l2s_ref, l2d_ref, m2_ref, b2_ref, out_ref):
    s1 = l2s_ref[:, :_L]
    s0 = l2s_ref[:, _L:]
    w = jnp.exp(_lrelu(s0 + l2d_ref[...]) - m2_ref[...])
    tot = (q0_ref[:, :_L] + q1_ref[:, :_L] + w * s1)
    rr = lax.broadcasted_iota(jnp.int32, (_L, _L), 0)
    sel0 = (rr == 0).astype(jnp.float32)
    sel1 = (rr == 1).astype(jnp.float32)
    numb = jnp.dot(tot, sel0, preferred_element_type=jnp.float32)
    denb = jnp.dot(tot, sel1, preferred_element_type=jnp.float32)
    o = numb / (denb + 1e-16) + b2_ref[...]
    out_ref[...] = jnp.where(o > 0, o, jnp.exp(jnp.minimum(o, 0.0)) - 1.0)


def _tc(body, out_shape, *ins):
    return pl.pallas_call(body, out_shape=out_shape)(*ins)


def _make_edge_pass(n_nodes, n_edges, row_w, n_vregs_payload, ch):
    ew = n_edges // _NW
    nch = ew // ch
    assert ew % ch == 0 and ch % 8 == 0 and ch <= 128
    npad = -(-n_nodes // (128 * _NS)) * (128 * _NS)
    rpt = npad // _NS
    assert rpt % ch == 0
    mesh = plsc.VectorSubcoreMesh(core_axis_name="c", subcore_axis_name="s",
                                  num_cores=_NC, num_subcores=_NS)

    nb = 5
    assert nch % nb == 0
    ngrp = nch // nb

    @functools.partial(
        pl.kernel,
        out_type=jax.ShapeDtypeStruct((_NC, npad, row_w), jnp.float32),
        mesh=mesh,
        compiler_params=pltpu.CompilerParams(use_tc_tiling_on_sc=False),
        scratch_types=[
            pltpu.VMEM((nb, ch), jnp.int32),
            pltpu.VMEM((nb, ch), jnp.int32),
            pltpu.VMEM((nb, ch, row_w), jnp.float32),
            pltpu.VMEM((nb, ch, _L), jnp.float32),
            pltpu.VMEM((_L,), jnp.float32),
            pltpu.VMEM_SHARED((npad, row_w), jnp.float32),
            pltpu.SemaphoreType.DMA,
            pltpu.SemaphoreType.DMA,
            pltpu.SemaphoreType.DMA,
        ],
    )
    def k(src_hbm, dst_hbm, pack_hbm, adp_hbm, m_hbm, out_hbm,
          idxs_v, idxd_v, rows_v, ad_v, m_v, acc_sh,
          semi, semg, sems):
        cid = lax.axis_index("c")
        sid = lax.axis_index("s")

        def zrow(r, carry):
            for kk in range(row_w // _L):
                rows_v[0, r, pl.ds(kk * _L, _L)] = jnp.zeros((_L,), jnp.float32)
            return carry
        lax.fori_loop(0, ch, zrow, 0)
        dz = []
        for b in range(rpt // ch):
            dz.append(pltpu.async_copy(
                rows_v.at[0], acc_sh.at[pl.ds(sid * rpt + b * ch, ch)], semi))
        for d in dz:
            d.wait()
        pltpu.sync_copy(m_hbm, m_v)
        plsc.subcore_barrier()

        mvec = m_v[...]
        ebase = (cid * _NS + sid) * ew

        def group(g, carry):
            base0 = ebase + g * (nb * ch)
            di = []
            for b in range(nb):
                di.append(pltpu.async_copy(
                    src_hbm.at[pl.ds(base0 + b * ch, ch)],
                    idxs_v.at[b], semi))
                di.append(pltpu.async_copy(
                    dst_hbm.at[pl.ds(base0 + b * ch, ch)],
                    idxd_v.at[b], semi))
            for d in di:
                d.wait()
            dg = []
            for b in range(nb):
                dg.append(pltpu.async_copy(
                    pack_hbm.at[idxs_v.at[b]], rows_v.at[b], semg))
                dg.append(pltpu.async_copy(
                    adp_hbm.at[idxd_v.at[b]], ad_v.at[b], semg))
            for d in dg:
                d.wait()
            dsc = []
            for b in range(nb):
                def edge(e2, c2, b=b):
                    asv = rows_v[b, e2, pl.ds(row_w - _L, _L)]
                    t = asv + ad_v[b, e2, pl.ds(0, _L)]
                    w = jnp.exp(jnp.maximum(t, _NEG * t) - mvec)
                    for kk in range(n_vregs_payload):
                        hv = rows_v[b, e2, pl.ds(kk * _L, _L)]
                        rows_v[b, e2, pl.ds(kk * _L, _L)] = hv * w
                    rows_v[b, e2, pl.ds(row_w - _L, _L)] = w
                    return c2
                lax.fori_loop(0, ch, edge, 0, unroll=4)
                dsc.append(pltpu.async_copy(
                    rows_v.at[b], acc_sh.at[idxd_v.at[b]], sems, add=True))
            for d in dsc:
                d.wait()
            return carry
        lax.fori_loop(0, ngrp, group, 0)

        plsc.subcore_barrier()
        pltpu.sync_copy(acc_sh.at[pl.ds(sid * rpt, rpt)],
                        out_hbm.at[cid, pl.ds(sid * rpt, rpt)])
    return k


def kernel(x, edge_index, W1, a1s, a1d, b1, W2, a2s, a2d, b2):
    n, f_in = x.shape
    e = edge_index.shape[1]
    h, f_hid = a1s.shape
    c1 = h * f_hid
    p1 = c1 + _L
    f32 = jnp.float32

    hh, ff = np.meshgrid(np.arange(h), np.arange(f_hid), indexing="ij")
    dest = ((ff // 2) * _L + (ff % 2) * h + hh).reshape(-1)
    inv = np.empty((c1,), np.int64)
    inv[dest] = np.arange(c1)

    w1flat = jnp.transpose(W1, (1, 0, 2)).reshape(f_in, c1)
    w1perm = w1flat[:, inv]
    avs = jnp.einsum("hif,hf->ih", W1, a1s)
    avd = jnp.einsum("hif,hf->ih", W1, a1d)
    wcat = jnp.concatenate(
        [w1perm, jnp.tile(avs, (1, 2)), jnp.tile(avd, (1, 2))], axis=1)

    b1p = b1.reshape(1, c1)[:, inv]
    w2p = W2[inv, :]
    w2rep = jnp.tile(w2p, (1, _L))
    wcat2 = jnp.concatenate(
        [w2rep, w2rep * a2s[0], w2rep * a2d[0]], axis=1)

    src = edge_index[0]
    dst = edge_index[1]

    hsrc, adp, m1 = _tc(
        _prep_body,
        (jax.ShapeDtypeStruct((n, p1), f32),
         jax.ShapeDtypeStruct((n, _L), f32),
         jax.ShapeDtypeStruct((1, _L), f32)),
        x, wcat)

    part1 = _make_edge_pass(n, e, p1, c1 // _L, 40)(
        src, dst, hsrc, adp, m1.reshape(_L))

    l2s, l2d, m2 = _tc(
        _mid_body,
        (jax.ShapeDtypeStruct((n, 2 * _L), f32),
         jax.ShapeDtypeStruct((n, _L), f32),
         jax.ShapeDtypeStruct((1, _L), f32)),
        part1[0, :n], part1[1, :n], hsrc, adp, m1, b1p, wcat2)

    part2 = _make_edge_pass(n, e, 2 * _L, 1, 80)(
        src, dst, l2s, l2d, m2.reshape(_L))

    out16 = _tc(
        _fin_body,
        jax.ShapeDtypeStruct((n, _L), f32),
        part2[0, :n], part2[1, :n], l2s, l2d, m2, b2.reshape(1, 1))

    return out16[:, :1]
